# Initial kernel scaffold; baseline (speedup 1.0000x reference)
import jax
import jax.numpy as jnp
from jax.experimental import pallas as pl


def kernel(relation_ids, table):
    raise NotImplementedError("write your pallas kernel here")


# speedup vs baseline: 5.2447x; 5.2447x over previous
import functools

import jax
import jax.numpy as jnp
from jax import lax
from jax.experimental import pallas as pl
from jax.experimental.pallas import tpu as pltpu
from jax.experimental.pallas import tpu_sc as plsc

_NUM_CORES = 2
_NUM_SUBCORES = 16
_NUM_WORKERS = _NUM_CORES * _NUM_SUBCORES

_CHUNK = 128


def _gather_kernel(n_chunks, chunk, ids_hbm, table_hbm, out_hbm,
                   idx_v, rows_v, gsem):
  wid = lax.axis_index("s") * _NUM_CORES + lax.axis_index("c")
  rows_per_worker = n_chunks * chunk
  base = wid * rows_per_worker

  pltpu.sync_copy(ids_hbm.at[wid], idx_v)

  @pl.loop(0, n_chunks)
  def _(j):
    pltpu.async_copy(table_hbm.at[idx_v.at[j]], rows_v, gsem).wait()
    pltpu.sync_copy(rows_v, out_hbm.at[pl.ds(base + j * chunk, chunk)])


def kernel(relation_ids, table):
  batch, hist = relation_ids.shape
  vocab, dim = table.shape
  total = batch * hist
  assert total % (_NUM_WORKERS * _CHUNK) == 0
  rows_per_worker = total // _NUM_WORKERS
  n_chunks = rows_per_worker // _CHUNK

  ids = relation_ids.reshape(_NUM_WORKERS, n_chunks, _CHUNK).astype(jnp.int32)

  mesh = plsc.VectorSubcoreMesh(core_axis_name="c", subcore_axis_name="s")
  grab = pl.kernel(
      functools.partial(_gather_kernel, n_chunks, _CHUNK),
      out_type=jax.ShapeDtypeStruct((total, dim), jnp.float32),
      mesh=mesh,
      scratch_types=[
          pltpu.VMEM((n_chunks, _CHUNK), jnp.int32),
          pltpu.VMEM((_CHUNK, dim), jnp.float32),
          pltpu.SemaphoreType.DMA,
      ],
      compiler_params=pltpu.CompilerParams(use_tc_tiling_on_sc=False),
  )
  out = grab(ids, table)
  return out.reshape(batch, hist, dim)


# speedup vs baseline: 6.2060x; 1.1833x over previous
import functools

import jax
import jax.numpy as jnp
from jax import lax
from jax.experimental import pallas as pl
from jax.experimental.pallas import tpu as pltpu
from jax.experimental.pallas import tpu_sc as plsc

_NUM_CORES = 2
_NUM_SUBCORES = 16
_NUM_WORKERS = _NUM_CORES * _NUM_SUBCORES

_CHUNK = 128
_NBUF = 4


def _gather_kernel(n_chunks, chunk, ids_hbm, table_hbm, out_hbm,
                   idx_v, rows_v, gsems, wsems):
  wid = lax.axis_index("s") * _NUM_CORES + lax.axis_index("c")
  rows_per_worker = n_chunks * chunk
  base = wid * rows_per_worker
  n_groups = n_chunks // _NBUF

  pltpu.sync_copy(ids_hbm.at[wid], idx_v)

  def start_gather(j, b):
    pltpu.async_copy(table_hbm.at[idx_v.at[j]], rows_v.at[b], gsems[b])

  def wait_gather(j, b):
    pltpu.make_async_copy(table_hbm.at[idx_v.at[j]], rows_v.at[b],
                          gsems[b]).wait()

  def start_write(j, b):
    pltpu.async_copy(rows_v.at[b], out_hbm.at[pl.ds(base + j * chunk, chunk)],
                     wsems[b])

  def wait_write(j, b):
    pltpu.make_async_copy(rows_v.at[b],
                          out_hbm.at[pl.ds(base + j * chunk, chunk)],
                          wsems[b]).wait()

  for b in range(_NBUF):
    start_gather(b, b)

  @pl.loop(0, n_groups - 1)
  def _(g):
    first = g * _NBUF
    for b in range(_NBUF):
      wait_gather(first + b, b)
      start_write(first + b, b)
    for b in range(_NBUF):
      wait_write(first + b, b)
      start_gather(first + _NBUF + b, b)

  last = (n_groups - 1) * _NBUF
  for b in range(_NBUF):
    wait_gather(last + b, b)
    start_write(last + b, b)
  for b in range(_NBUF):
    wait_write(last + b, b)


def kernel(relation_ids, table):
  batch, hist = relation_ids.shape
  vocab, dim = table.shape
  total = batch * hist
  assert total % (_NUM_WORKERS * _CHUNK * _NBUF) == 0
  rows_per_worker = total // _NUM_WORKERS
  n_chunks = rows_per_worker // _CHUNK

  ids = relation_ids.reshape(_NUM_WORKERS, n_chunks, _CHUNK).astype(jnp.int32)

  mesh = plsc.VectorSubcoreMesh(core_axis_name="c", subcore_axis_name="s")
  grab = pl.kernel(
      functools.partial(_gather_kernel, n_chunks, _CHUNK),
      out_type=jax.ShapeDtypeStruct((total, dim), jnp.float32),
      mesh=mesh,
      scratch_types=[
          pltpu.VMEM((n_chunks, _CHUNK), jnp.int32),
          pltpu.VMEM((_NBUF, _CHUNK, dim), jnp.float32),
          [pltpu.SemaphoreType.DMA] * _NBUF,
          [pltpu.SemaphoreType.DMA] * _NBUF,
      ],
      compiler_params=pltpu.CompilerParams(use_tc_tiling_on_sc=False),
  )
  out = grab(ids, table)
  return out.reshape(batch, hist, dim)
